# baseline (speedup 1.0000x reference)
import dataclasses
import functools

import jax
import jax.numpy as jnp
from jax import lax
from jax.experimental import pallas as pl
from jax.experimental.pallas import tpu as pltpu
from jax.experimental.pallas import tpu_sc as plsc

N_NODES = 100001
D = 256
B = 16384

NC = 2
NS = 16
NW = NC * NS
BPW = B // NW
RNG = 3200
DUMP = 100000

_mesh = plsc.VectorSubcoreMesh(core_axis_name="c", subcore_axis_name="s")

_sc_params = pltpu.CompilerParams()
if "needs_layout_passes" in pltpu.CompilerParams.__dataclass_fields__:
    _sc_params = dataclasses.replace(_sc_params, needs_layout_passes=False)


def _wid():
    return lax.axis_index("s") * NC + lax.axis_index("c")


@functools.partial(
    pl.kernel,
    mesh=_mesh,
    out_type=jax.ShapeDtypeStruct((B, D), jnp.float32),
    scratch_types=[
        pltpu.VMEM((4, 128), jnp.int32),
        pltpu.VMEM((128, D), jnp.float32),
        pltpu.VMEM((128, D), jnp.float32),
        pltpu.SemaphoreType.DMA,
        pltpu.SemaphoreType.DMA,
        pltpu.SemaphoreType.DMA,
        pltpu.SemaphoreType.DMA,
    ],
)
def _sc_gather(mem_hbm, idx_hbm, h_hbm, idx_v, buf0, buf1, g0, g1, s0, s1):
    wid = _wid()
    base = wid * BPW
    pltpu.sync_copy(idx_hbm.at[pl.ds(wid * 4, 4)], idx_v)
    bufs = (buf0, buf1)
    gsems = (g0, g1)
    ssems = (s0, s1)

    def _gather(j):
        return pltpu.async_copy(mem_hbm.at[idx_v.at[j]], bufs[j % 2],
                                gsems[j % 2])

    def _writeout(j):
        return pltpu.async_copy(bufs[j % 2],
                                h_hbm.at[pl.ds(base + j * 128, 128)],
                                ssems[j % 2])

    gd = [_gather(0), _gather(1)]
    gd[0].wait()
    wd0 = _writeout(0)
    gd[1].wait()
    wd1 = _writeout(1)
    wd0.wait()
    gd2 = _gather(2)
    wd1.wait()
    gd3 = _gather(3)
    gd2.wait()
    wd0 = _writeout(2)
    gd3.wait()
    wd1 = _writeout(3)
    wd0.wait()
    wd1.wait()


@functools.partial(
    pl.kernel,
    mesh=_mesh,
    out_type=jax.ShapeDtypeStruct((NW * 1024,), jnp.int32),
    scratch_types=[
        pltpu.VMEM((128, 128), jnp.int32),
        pltpu.VMEM((RNG,), jnp.int32),
        pltpu.VMEM((1024,), jnp.int32),
    ],
    compiler_params=_sc_params,
)
def _sc_dedupe(idx_hbm, fl_hbm, idx_v, wtab_v, fl_v):
    wid = _wid()
    base = wid * RNG
    pltpu.sync_copy(idx_hbm, idx_v)
    lanes = lax.iota(jnp.int32, 16)

    @pl.loop(0, 128)
    def _(r):
        @pl.loop(0, 8)
        def _(k):
            idxc = idx_v[r, pl.ds(k * 16, 16)]
            ivec = (r * 128 + k * 16) + lanes
            _, last_m = plsc.scan_count(idxc)
            local = idxc - base
            inr = (local >= 0) & (local < RNG)
            m = last_m & inr
            localc = jnp.minimum(jnp.maximum(local, 0), RNG - 1)
            plsc.store_scatter(wtab_v, [localc], ivec, mask=m)

    @pl.loop(0, 64)
    def _(g):
        acc = lanes * 0
        for c in range(16):
            r = g * 2 + (c >> 3)
            idxc = idx_v[r, pl.ds((c & 7) * 16, 16)]
            ivec = (g * 256 + c * 16) + lanes
            local = idxc - base
            inr = (local >= 0) & (local < RNG)
            localc = jnp.minimum(jnp.maximum(local, 0), RNG - 1)
            wv = plsc.load_gather(wtab_v, [localc])
            m = inr & (wv == ivec)
            acc = acc | jnp.where(m, 1 << c, 0)
        fl_v[pl.ds(g * 16, 16)] = acc

    pltpu.sync_copy(fl_v, fl_hbm.at[pl.ds(wid * 1024, 1024)])


def _gru_body(val_ref, h_ref, W_ref, Wih_ref, Whh_ref, bih_ref, bhh_ref,
              out_ref):
    val = val_ref[...]
    h = h_ref[...]
    prec = jax.lax.Precision.DEFAULT
    msg = jax.lax.dot_general(val, W_ref[...], (((1,), (0,)), ((), ())),
                              precision=prec)
    gi = jax.lax.dot_general(msg, Wih_ref[...], (((1,), (1,)), ((), ())),
                             precision=prec) + bih_ref[...][None, :]
    gh = jax.lax.dot_general(h, Whh_ref[...], (((1,), (1,)), ((), ())),
                             precision=prec) + bhh_ref[...][None, :]
    i_r = gi[:, :D]
    i_z = gi[:, D:2 * D]
    i_n = gi[:, 2 * D:]
    h_r = gh[:, :D]
    h_z = gh[:, D:2 * D]
    h_n = gh[:, 2 * D:]
    r = jax.nn.sigmoid(i_r + h_r)
    z = jax.nn.sigmoid(i_z + h_z)
    n = jnp.tanh(i_n + r * h_n)
    out_ref[...] = (1.0 - z) * n + z * h


@functools.partial(
    pl.kernel,
    mesh=_mesh,
    out_type=(),
    scratch_types=[
        pltpu.VMEM((4, 128), jnp.int32),
        pltpu.VMEM((NW * 1024,), jnp.int32),
        pltpu.VMEM((4, 128), jnp.int32),
        pltpu.VMEM((128, D), jnp.float32),
        pltpu.VMEM((128, D), jnp.float32),
        pltpu.SemaphoreType.DMA,
        pltpu.SemaphoreType.DMA,
        pltpu.SemaphoreType.DMA,
        pltpu.SemaphoreType.DMA,
        pltpu.SemaphoreType.DMA,
    ],
)
def _sc_scatter(idx_hbm, fl_hbm, hnew_hbm, out_ref, idx_v, fl_v,
                tgt_v, buf0, buf1, wsem, g0, g1, s0, s1):
    wid = _wid()
    base = wid * BPW
    lanes = lax.iota(jnp.int32, 16)
    pltpu.sync_copy(idx_hbm.at[pl.ds(wid * 4, 4)], idx_v)
    fld = pltpu.async_copy(fl_hbm, fl_v, wsem)

    bufs = (buf0, buf1)
    gsems = (g0, g1)
    ssems = (s0, s1)

    def _gather(j):
        return pltpu.async_copy(hnew_hbm.at[pl.ds(base + j * 128, 128)],
                                bufs[j % 2], gsems[j % 2])

    def _scatter(j):
        return pltpu.async_copy(bufs[j % 2], out_ref.at[tgt_v.at[j]],
                                ssems[j % 2])

    gd = [_gather(0), _gather(1)]
    fld.wait()

    accs = []
    for gg in range(2):
        acc = lanes * 0
        for dw in range(NW):
            acc = acc | fl_v[pl.ds(dw * 1024 + (wid * 2 + gg) * 16, 16)]
        accs.append(acc)

    def _tgt(j):
        for k in range(8):
            lc = j * 8 + k
            idxc = idx_v[j, pl.ds(k * 16, 16)]
            winner = ((accs[lc >> 4] >> (lc & 15)) & 1) == 1
            tgt_v[j, pl.ds(k * 16, 16)] = jnp.where(winner, idxc, DUMP)

    for j in range(4):
        _tgt(j)

    gd[0].wait()
    sd0 = _scatter(0)
    gd[1].wait()
    sd1 = _scatter(1)
    sd0.wait()
    gd2 = _gather(2)
    sd1.wait()
    gd3 = _gather(3)
    gd2.wait()
    sd0 = _scatter(2)
    gd3.wait()
    sd1 = _scatter(3)
    sd0.wait()
    sd1.wait()


@functools.partial(
    pl.kernel,
    mesh=_mesh,
    out_type=(),
    scratch_types=[
        pltpu.VMEM((1, D), jnp.float32),
    ],
)
def _sc_repair(mem_hbm, out_ref, row_v):
    wid = _wid()

    @pl.when(wid == 0)
    def _():
        pltpu.sync_copy(mem_hbm.at[pl.ds(DUMP, 1)], row_v)
        pltpu.sync_copy(row_v, out_ref.at[pl.ds(DUMP, 1)])


def kernel(mem, idx, val, W, W_ih, W_hh, b_ih, b_hh):
    idx2 = idx.astype(jnp.int32).reshape(128, 128)

    h = _sc_gather(mem, idx2)
    w_arr = _sc_dedupe(idx2)

    BM = 1024
    n_blocks = B // BM
    h_new = pl.pallas_call(
        _gru_body,
        grid=(n_blocks,),
        in_specs=[
            pl.BlockSpec((BM, D), lambda i: (i, 0)),
            pl.BlockSpec((BM, D), lambda i: (i, 0)),
            pl.BlockSpec((D, D), lambda i: (0, 0)),
            pl.BlockSpec((3 * D, D), lambda i: (0, 0)),
            pl.BlockSpec((3 * D, D), lambda i: (0, 0)),
            pl.BlockSpec((3 * D,), lambda i: (0,)),
            pl.BlockSpec((3 * D,), lambda i: (0,)),
        ],
        out_specs=pl.BlockSpec((BM, D), lambda i: (i, 0)),
        out_shape=jax.ShapeDtypeStruct((B, D), jnp.float32),
    )(val, h, W, W_ih, W_hh, b_ih, b_hh)

    out_ref = jax.new_ref(mem)
    _sc_scatter(idx2, w_arr, h_new, out_ref)
    _sc_repair(mem, out_ref)
    return jax.freeze(out_ref)


# speedup vs baseline: 1.0232x; 1.0232x over previous
import dataclasses
import functools

import jax
import jax.numpy as jnp
from jax import lax
from jax.experimental import pallas as pl
from jax.experimental.pallas import tpu as pltpu
from jax.experimental.pallas import tpu_sc as plsc

N_NODES = 100001
D = 256
B = 16384

NC = 2
NS = 16
NW = NC * NS
BPW = B // NW
RNG = 3128
WPAD = NW * RNG
DUMP = 100000

_mesh = plsc.VectorSubcoreMesh(core_axis_name="c", subcore_axis_name="s")

_sc_params = pltpu.CompilerParams()
if "needs_layout_passes" in pltpu.CompilerParams.__dataclass_fields__:
    _sc_params = dataclasses.replace(_sc_params, needs_layout_passes=False)


def _wid():
    return lax.axis_index("s") * NC + lax.axis_index("c")


@functools.partial(
    pl.kernel,
    mesh=_mesh,
    out_type=jax.ShapeDtypeStruct((B, D), jnp.float32),
    scratch_types=[
        pltpu.VMEM((4, 128), jnp.int32),
        pltpu.VMEM((128, D), jnp.float32),
        pltpu.VMEM((128, D), jnp.float32),
        pltpu.SemaphoreType.DMA,
        pltpu.SemaphoreType.DMA,
        pltpu.SemaphoreType.DMA,
        pltpu.SemaphoreType.DMA,
    ],
)
def _sc_gather(mem_hbm, idx_hbm, h_hbm, idx_v, buf0, buf1, g0, g1, s0, s1):
    wid = _wid()
    base = wid * BPW
    pltpu.sync_copy(idx_hbm.at[pl.ds(wid * 4, 4)], idx_v)
    bufs = (buf0, buf1)
    gsems = (g0, g1)
    ssems = (s0, s1)

    def _gather(j):
        return pltpu.async_copy(mem_hbm.at[idx_v.at[j]], bufs[j % 2],
                                gsems[j % 2])

    def _writeout(j):
        return pltpu.async_copy(bufs[j % 2],
                                h_hbm.at[pl.ds(base + j * 128, 128)],
                                ssems[j % 2])

    gd = [_gather(0), _gather(1)]
    gd[0].wait()
    wd0 = _writeout(0)
    gd[1].wait()
    wd1 = _writeout(1)
    wd0.wait()
    gd2 = _gather(2)
    wd1.wait()
    gd3 = _gather(3)
    gd2.wait()
    wd0 = _writeout(2)
    gd3.wait()
    wd1 = _writeout(3)
    wd0.wait()
    wd1.wait()


@functools.partial(
    pl.kernel,
    mesh=_mesh,
    out_type=jax.ShapeDtypeStruct((WPAD,), jnp.int32),
    scratch_types=[
        pltpu.VMEM((128, 128), jnp.int32),
        pltpu.VMEM((RNG,), jnp.int32),
    ],
    compiler_params=_sc_params,
)
def _sc_dedupe(idx_hbm, w_hbm, idx_v, wtab_v):
    wid = _wid()
    base = wid * RNG
    pltpu.sync_copy(idx_hbm, idx_v)
    lanes = lax.iota(jnp.int32, 16)

    @pl.loop(0, 128)
    def _(r):
        @pl.loop(0, 8)
        def _(k):
            idxc = idx_v[r, pl.ds(k * 16, 16)]
            ivec = (r * 128 + k * 16) + lanes
            _, last_m = plsc.scan_count(idxc)
            local = idxc - base
            inr = (local >= 0) & (local < RNG)
            m = last_m & inr
            localc = jnp.minimum(jnp.maximum(local, 0), RNG - 1)
            plsc.store_scatter(wtab_v, [localc], ivec, mask=m)

    pltpu.sync_copy(wtab_v, w_hbm.at[pl.ds(base, RNG)])


def _gru_body(val_ref, h_ref, W_ref, Wih_ref, Whh_ref, bih_ref, bhh_ref,
              out_ref):
    val = val_ref[...]
    h = h_ref[...]
    prec = jax.lax.Precision.DEFAULT
    msg = jax.lax.dot_general(val, W_ref[...], (((1,), (0,)), ((), ())),
                              precision=prec)
    gi = jax.lax.dot_general(msg, Wih_ref[...], (((1,), (1,)), ((), ())),
                             precision=prec) + bih_ref[...][None, :]
    gh = jax.lax.dot_general(h, Whh_ref[...], (((1,), (1,)), ((), ())),
                             precision=prec) + bhh_ref[...][None, :]
    i_r = gi[:, :D]
    i_z = gi[:, D:2 * D]
    i_n = gi[:, 2 * D:]
    h_r = gh[:, :D]
    h_z = gh[:, D:2 * D]
    h_n = gh[:, 2 * D:]
    r = jax.nn.sigmoid(i_r + h_r)
    z = jax.nn.sigmoid(i_z + h_z)
    n = jnp.tanh(i_n + r * h_n)
    out_ref[...] = (1.0 - z) * n + z * h


@functools.partial(
    pl.kernel,
    mesh=_mesh,
    out_type=(),
    scratch_types=[
        pltpu.VMEM((4, 128), jnp.int32),
        pltpu.VMEM((4, 128), jnp.int32),
        pltpu.VMEM((4, 128), jnp.int32),
        pltpu.VMEM((128, D), jnp.float32),
        pltpu.VMEM((128, D), jnp.float32),
        pltpu.SemaphoreType.DMA,
        pltpu.SemaphoreType.DMA,
        pltpu.SemaphoreType.DMA,
        pltpu.SemaphoreType.DMA,
        pltpu.SemaphoreType.DMA,
    ],
)
def _sc_scatter(idx_hbm, w_hbm, hnew_hbm, out_ref, idx_v, wv_v, tgt_v,
                buf0, buf1, wsem, g0, g1, s0, s1):
    wid = _wid()
    base = wid * BPW
    pltpu.sync_copy(idx_hbm.at[pl.ds(wid * 4, 4)], idx_v)

    wvd = [pltpu.async_copy(w_hbm.at[idx_v.at[j]], wv_v.at[j], wsem)
           for j in range(4)]

    bufs = (buf0, buf1)
    gsems = (g0, g1)
    ssems = (s0, s1)

    def _gather(j):
        return pltpu.async_copy(hnew_hbm.at[pl.ds(base + j * 128, 128)],
                                bufs[j % 2], gsems[j % 2])

    def _scatter(j):
        return pltpu.async_copy(bufs[j % 2], out_ref.at[tgt_v.at[j]],
                                ssems[j % 2])

    gd = [_gather(0), _gather(1)]
    for d in wvd:
        d.wait()

    lanes = lax.iota(jnp.int32, 16)
    for j in range(4):
        for k in range(8):
            idxc = idx_v[j, pl.ds(k * 16, 16)]
            wvc = wv_v[j, pl.ds(k * 16, 16)]
            ivec = (base + j * 128 + k * 16) + lanes
            winner = wvc == ivec
            tgt_v[j, pl.ds(k * 16, 16)] = jnp.where(winner, idxc, DUMP)

    gd[0].wait()
    sd0 = _scatter(0)
    gd[1].wait()
    sd1 = _scatter(1)
    sd0.wait()
    gd2 = _gather(2)
    sd1.wait()
    gd3 = _gather(3)
    gd2.wait()
    sd0 = _scatter(2)
    gd3.wait()
    sd1 = _scatter(3)
    sd0.wait()
    sd1.wait()


@functools.partial(
    pl.kernel,
    mesh=_mesh,
    out_type=(),
    scratch_types=[
        pltpu.VMEM((1, D), jnp.float32),
    ],
)
def _sc_repair(mem_hbm, out_ref, row_v):
    wid = _wid()

    @pl.when(wid == 0)
    def _():
        pltpu.sync_copy(mem_hbm.at[pl.ds(DUMP, 1)], row_v)
        pltpu.sync_copy(row_v, out_ref.at[pl.ds(DUMP, 1)])


def kernel(mem, idx, val, W, W_ih, W_hh, b_ih, b_hh):
    idx2 = idx.astype(jnp.int32).reshape(128, 128)

    h = _sc_gather(mem, idx2)
    w_arr = _sc_dedupe(idx2)

    BM = 1024
    n_blocks = B // BM
    h_new = pl.pallas_call(
        _gru_body,
        grid=(n_blocks,),
        in_specs=[
            pl.BlockSpec((BM, D), lambda i: (i, 0)),
            pl.BlockSpec((BM, D), lambda i: (i, 0)),
            pl.BlockSpec((D, D), lambda i: (0, 0)),
            pl.BlockSpec((3 * D, D), lambda i: (0, 0)),
            pl.BlockSpec((3 * D, D), lambda i: (0, 0)),
            pl.BlockSpec((3 * D,), lambda i: (0,)),
            pl.BlockSpec((3 * D,), lambda i: (0,)),
        ],
        out_specs=pl.BlockSpec((BM, D), lambda i: (i, 0)),
        out_shape=jax.ShapeDtypeStruct((B, D), jnp.float32),
    )(val, h, W, W_ih, W_hh, b_ih, b_hh)

    out_ref = jax.new_ref(mem)
    _sc_scatter(idx2, w_arr, h_new, out_ref)
    _sc_repair(mem, out_ref)
    return jax.freeze(out_ref)
